# baseline (speedup 1.0000x reference)
import jax
import jax.numpy as jnp
from jax import lax
from jax.experimental import pallas as pl
from jax.experimental.pallas import tpu as pltpu
from jax.experimental.pallas import tpu_sc as plsc

D_HALF_K = 384
H_K = 32
W_K = 32
N_K = H_K * W_K
NW_K = 32
HR_PER_W_K = 2 * N_K // NW_K


def _sc_body(table_hbm, idx_hbm, out_hbm, idx_v, rows_v, sem):
    wid = lax.axis_index("s") * 2 + lax.axis_index("c")
    base = wid * HR_PER_W_K
    pltpu.sync_copy(idx_hbm.at[pl.ds(base, HR_PER_W_K)], idx_v)
    pltpu.async_copy(table_hbm.at[idx_v], rows_v, sem).wait()
    pltpu.sync_copy(rows_v, out_hbm.at[pl.ds(base, HR_PER_W_K)])


def kernel(h, w, row_table, col_table):
    table = jnp.concatenate([row_table, col_table], axis=0)
    n = jnp.arange(N_K, dtype=jnp.int32)
    hm1 = jnp.asarray(h, jnp.int32) - 1
    wm1 = jnp.asarray(w, jnp.int32) - 1
    idx_row = jnp.minimum(n // W_K, hm1)
    idx_col = row_table.shape[0] + jnp.minimum(n % W_K, wm1)
    idx = jnp.stack([idx_row, idx_col], axis=1).reshape(2 * N_K)
    k = pl.kernel(
        _sc_body,
        mesh=plsc.VectorSubcoreMesh(core_axis_name="c", subcore_axis_name="s"),
        out_type=jax.ShapeDtypeStruct((2 * N_K, D_HALF_K), jnp.float32),
        scratch_types=[
            pltpu.VMEM((HR_PER_W_K,), jnp.int32),
            pltpu.VMEM((HR_PER_W_K, D_HALF_K), jnp.float32),
            pltpu.SemaphoreType.DMA,
        ],
    )
    return k(table, idx).reshape(N_K, 2 * D_HALF_K)


# speedup vs baseline: 1.1979x; 1.1979x over previous
import jax
import jax.numpy as jnp
from jax import lax
from jax.experimental import pallas as pl
from jax.experimental.pallas import tpu as pltpu
from jax.experimental.pallas import tpu_sc as plsc

D_HALF_K = 384
H_K = 32
W_K = 32
N_K = H_K * W_K
B_K = 32


def _sc_body(row_hbm, col_hbm, hw_hbm, out_hbm, hw_v, rows_v, sem_hw, sem_g):
    wid = lax.axis_index("s") * 2 + lax.axis_index("c")
    base = wid * B_K
    cp_hw = pltpu.async_copy(hw_hbm, hw_v, sem_hw)
    iota = lax.iota(jnp.int32, 16)
    widv = jnp.broadcast_to(wid, (16,)).astype(jnp.int32)
    g0 = pltpu.async_copy(row_hbm.at[widv], rows_v.at[pl.ds(0, 16)], sem_g)
    g1 = pltpu.async_copy(row_hbm.at[widv], rows_v.at[pl.ds(16, 16)], sem_g)
    g2 = pltpu.async_copy(col_hbm.at[iota], rows_v.at[pl.ds(32, 16)], sem_g)
    g3 = pltpu.async_copy(col_hbm.at[iota + 16], rows_v.at[pl.ds(48, 16)], sem_g)
    cp_hw.wait()
    hwv = hw_v[...]
    hm1 = hwv[0] - 1
    wm1 = hwv[1] - 1
    g0.wait()
    g1.wait()
    g2.wait()
    g3.wait()

    @pl.when((hm1 < B_K - 1) | (wm1 < B_K - 1))
    def _reclamped():
        idx_r = jnp.minimum(widv, jnp.maximum(hm1, 0))
        idx_c0 = jnp.minimum(iota, jnp.maximum(wm1, 0))
        idx_c1 = jnp.minimum(iota + 16, jnp.maximum(wm1, 0))
        pltpu.async_copy(row_hbm.at[idx_r], rows_v.at[pl.ds(0, 16)], sem_g).wait()
        pltpu.async_copy(row_hbm.at[idx_r], rows_v.at[pl.ds(16, 16)], sem_g).wait()
        pltpu.async_copy(col_hbm.at[idx_c0], rows_v.at[pl.ds(32, 16)], sem_g).wait()
        pltpu.async_copy(col_hbm.at[idx_c1], rows_v.at[pl.ds(48, 16)], sem_g).wait()

    pltpu.sync_copy(rows_v.at[pl.ds(0, B_K)],
                    out_hbm.at[pl.ds(base, B_K), pl.ds(0, D_HALF_K)])
    pltpu.sync_copy(rows_v.at[pl.ds(B_K, B_K)],
                    out_hbm.at[pl.ds(base, B_K), pl.ds(D_HALF_K, D_HALF_K)])


def kernel(h, w, row_table, col_table):
    hw8 = jnp.zeros((16,), jnp.int32).at[0].set(h).at[1].set(w)
    k = pl.kernel(
        _sc_body,
        mesh=plsc.VectorSubcoreMesh(core_axis_name="c", subcore_axis_name="s"),
        out_type=jax.ShapeDtypeStruct((N_K, 2 * D_HALF_K), jnp.float32),
        scratch_types=[
            pltpu.VMEM((16,), jnp.int32),
            pltpu.VMEM((2 * B_K, D_HALF_K), jnp.float32),
            pltpu.SemaphoreType.DMA,
            pltpu.SemaphoreType.DMA,
        ],
    )
    return k(row_table, col_table, hw8)
